# baseline (speedup 1.0000x reference)
import functools

import jax
import jax.numpy as jnp
from jax import lax
from jax.experimental import pallas as pl
from jax.experimental.pallas import tpu as pltpu
from jax.experimental.pallas import tpu_sc as plsc

D = 32
L = 16
NC = 2
NS = 16
NW = NC * NS
ICH = 128
BCH = 128


def _sci_body(b_per_w, sid_hbm, semb_hbm, srows_hbm, sidx_v, rows_v, sem):
    n_ch = b_per_w // ICH
    wid = lax.axis_index("s") * NC + lax.axis_index("c")
    base = pl.multiple_of(wid * b_per_w, b_per_w)

    for c in range(n_ch):
        pltpu.sync_copy(sid_hbm.at[pl.ds(base + c * ICH, ICH)], sidx_v.at[c])
    copies = [
        pltpu.async_copy(semb_hbm.at[sidx_v.at[c]],
                         rows_v.at[pl.ds(c * ICH, ICH)], sem)
        for c in range(n_ch)
    ]
    for cp in copies:
        cp.wait()
    pltpu.sync_copy(rows_v, srows_hbm.at[pl.ds(base, b_per_w)])


def _dot_body(b_per_w, pid_hbm, pemb_hbm, srows_hbm, out_hbm,
              pidx_v, srows_v, prows_v, out_v, sem0, sem1):
    n_big = b_per_w // BCH
    wid = lax.axis_index("s") * NC + lax.axis_index("c")
    base = pl.multiple_of(wid * b_per_w, b_per_w)

    pltpu.sync_copy(pid_hbm.at[pl.ds(base, b_per_w)], pidx_v)
    pltpu.sync_copy(srows_hbm.at[pl.ds(base, b_per_w)], srows_v)

    lane = lax.iota(jnp.int32, L)
    last_lane = lane == (L - 1)

    def big_chunk(c, carry):
        b0 = pl.multiple_of(c * BCH, BCH)

        def fire(f, carry2):
            r0 = pl.multiple_of(f * L, L)
            pvec = pidx_v[pl.ds(b0 + r0, L)]
            for u in range(L):
                pltpu.async_copy(pemb_hbm.at[pl.ds(pvec[u], 1)],
                                 prows_v.at[pl.ds(r0 + u, 1)],
                                 sem0 if u % 2 == 0 else sem1)
            return carry2

        lax.fori_loop(0, BCH // L, fire, 0)

        pltpu.make_async_copy(pemb_hbm.at[pl.ds(0, BCH // 2)],
                              prows_v.at[pl.ds(0, BCH // 2)], sem0).wait()
        pltpu.make_async_copy(pemb_hbm.at[pl.ds(0, BCH // 2)],
                              prows_v.at[pl.ds(0, BCH // 2)], sem1).wait()

        def compute(g, carry2):
            r0 = pl.multiple_of(g * L, L)
            for u in range(L):
                q = (srows_v[b0 + r0 + u, pl.ds(0, L)]
                     * prows_v[r0 + u, pl.ds(0, L)]
                     + srows_v[b0 + r0 + u, pl.ds(L, L)]
                     * prows_v[r0 + u, pl.ds(L, L)])
                cum = plsc.cumsum(q)
                plsc.store_scatter(
                    out_v, [jnp.full((L,), b0 + r0 + u, jnp.int32)],
                    cum, mask=last_lane)
            return carry2

        lax.fori_loop(0, BCH // L, compute, 0)
        return carry

    lax.fori_loop(0, n_big, big_chunk, 0)
    pltpu.sync_copy(out_v, out_hbm.at[pl.ds(base, b_per_w)])


def kernel(sid, pid, scientist_emb, paper_emb):
    batch = sid.shape[0]
    b_per_w = batch // NW
    mesh = plsc.VectorSubcoreMesh(core_axis_name="c", subcore_axis_name="s",
                                  num_cores=NC, num_subcores=NS)

    gather_sci = pl.kernel(
        functools.partial(_sci_body, b_per_w),
        out_type=jax.ShapeDtypeStruct((batch, D), jnp.float32),
        mesh=mesh,
        scratch_types=[
            pltpu.VMEM((b_per_w // ICH, ICH), jnp.int32),
            pltpu.VMEM((b_per_w, D), jnp.float32),
            pltpu.SemaphoreType.DMA,
        ],
        compiler_params=pltpu.CompilerParams(needs_layout_passes=False,
                                             use_tc_tiling_on_sc=False),
    )
    s_rows = gather_sci(sid.astype(jnp.int32), scientist_emb)

    dot = pl.kernel(
        functools.partial(_dot_body, b_per_w),
        out_type=jax.ShapeDtypeStruct((batch,), jnp.float32),
        mesh=mesh,
        scratch_types=[
            pltpu.VMEM((b_per_w,), jnp.int32),
            pltpu.VMEM((b_per_w, D), jnp.float32),
            pltpu.VMEM((BCH, D), jnp.float32),
            pltpu.VMEM((b_per_w,), jnp.float32),
            pltpu.SemaphoreType.DMA,
            pltpu.SemaphoreType.DMA,
        ],
        compiler_params=pltpu.CompilerParams(needs_layout_passes=False,
                                             use_tc_tiling_on_sc=True),
    )
    return dot(pid.astype(jnp.int32), paper_emb, s_rows)


# speedup vs baseline: 1.0904x; 1.0904x over previous
import functools

import jax
import jax.numpy as jnp
from jax import lax
from jax.experimental import pallas as pl
from jax.experimental.pallas import tpu as pltpu
from jax.experimental.pallas import tpu_sc as plsc

D = 32
L = 16
NC = 2
NS = 16
NW = NC * NS
SCH = 256


def _dot_body(b_per_w, sid_hbm, pid_hbm, semb_hbm, pemb_hbm, out_hbm,
              sidx_v, pidx_v, srows_v, prows_v, out_v, *sems):
    wid = lax.axis_index("s") * NC + lax.axis_index("c")
    base = pl.multiple_of(wid * b_per_w, b_per_w)

    pltpu.sync_copy(sid_hbm.at[pl.ds(base, b_per_w)], sidx_v)
    pltpu.sync_copy(pid_hbm.at[pl.ds(base, b_per_w)], pidx_v)

    lane = lax.iota(jnp.int32, L)
    last_lane = lane == (L - 1)

    def fire(idx_ref, table_hbm, rows_v, i0, n, sem_a, sem_b):
        def go(f, carry):
            r0 = pl.multiple_of(f * L, L)
            vec = idx_ref[pl.ds(i0 + r0, L)]
            for u in range(L):
                pltpu.async_copy(table_hbm.at[pl.ds(vec[u], 1)],
                                 rows_v.at[pl.ds(r0 + u, 1)],
                                 sem_a if u % 2 == 0 else sem_b)
            return carry
        lax.fori_loop(0, n // L, go, 0)

    def drain(table_hbm, rows_v, n, sem_a, sem_b):
        pltpu.make_async_copy(table_hbm.at[pl.ds(0, n // 2)],
                              rows_v.at[pl.ds(0, n // 2)], sem_a).wait()
        pltpu.make_async_copy(table_hbm.at[pl.ds(0, n // 2)],
                              rows_v.at[pl.ds(0, n // 2)], sem_b).wait()

    fire(pidx_v, pemb_hbm, prows_v, 0, b_per_w, sems[0], sems[1])

    def half(h, carry):
        h0 = pl.multiple_of(h * SCH, SCH)
        fire(sidx_v, semb_hbm, srows_v, h0, SCH, sems[2], sems[3])
        drain(semb_hbm, srows_v, SCH, sems[2], sems[3])

        @pl.when(h == 0)
        def _():
            drain(pemb_hbm, prows_v, b_per_w, sems[0], sems[1])

        def compute(g, carry2):
            r0 = pl.multiple_of(g * L, L)
            for u in range(L):
                q = (srows_v[r0 + u, pl.ds(0, L)]
                     * prows_v[h0 + r0 + u, pl.ds(0, L)]
                     + srows_v[r0 + u, pl.ds(L, L)]
                     * prows_v[h0 + r0 + u, pl.ds(L, L)])
                cum = plsc.cumsum(q)
                plsc.store_scatter(
                    out_v, [jnp.full((L,), h0 + r0 + u, jnp.int32)],
                    cum, mask=last_lane)
            return carry2

        lax.fori_loop(0, SCH // L, compute, 0)
        return carry

    lax.fori_loop(0, b_per_w // SCH, half, 0)
    pltpu.sync_copy(out_v, out_hbm.at[pl.ds(base, b_per_w)])


def kernel(sid, pid, scientist_emb, paper_emb):
    batch = sid.shape[0]
    b_per_w = batch // NW
    mesh = plsc.VectorSubcoreMesh(core_axis_name="c", subcore_axis_name="s",
                                  num_cores=NC, num_subcores=NS)
    k = pl.kernel(
        functools.partial(_dot_body, b_per_w),
        out_type=jax.ShapeDtypeStruct((batch,), jnp.float32),
        mesh=mesh,
        scratch_types=[
            pltpu.VMEM((b_per_w,), jnp.int32),
            pltpu.VMEM((b_per_w,), jnp.int32),
            pltpu.VMEM((SCH, D), jnp.float32),
            pltpu.VMEM((b_per_w, D), jnp.float32),
            pltpu.VMEM((b_per_w,), jnp.float32),
        ] + [pltpu.SemaphoreType.DMA] * 4,
        compiler_params=pltpu.CompilerParams(needs_layout_passes=False,
                                             use_tc_tiling_on_sc=True),
    )
    return k(sid.astype(jnp.int32), pid.astype(jnp.int32),
             scientist_emb, paper_emb)
